# baseline (speedup 1.0000x reference)
import jax
import jax.numpy as jnp
from jax.experimental import pallas as pl
from jax.experimental.pallas import tpu as pltpu

TOK_BLK = 512


def _router_kernel(x_ref, w_ref, o_ref):
    o_ref[0] = jax.lax.dot_general(
        w_ref[...], x_ref[0],
        dimension_numbers=(((1,), (1,)), ((), ())),
        preferred_element_type=jnp.float32)


def kernel(hidden_states, W):
    B, S, H = hidden_states.shape
    E = W.shape[0]
    out_t = pl.pallas_call(
        _router_kernel,
        grid=(B, S // TOK_BLK),
        in_specs=[
            pl.BlockSpec((1, TOK_BLK, H), lambda b, i: (b, i, 0)),
            pl.BlockSpec((E, H), lambda b, i: (0, 0)),
        ],
        out_specs=pl.BlockSpec((1, E, TOK_BLK), lambda b, i: (b, 0, i)),
        out_shape=jax.ShapeDtypeStruct((B, E, S), jnp.float32),
        compiler_params=pltpu.CompilerParams(
            dimension_semantics=("arbitrary", "arbitrary"),
        ),
    )(hidden_states, W)
    return jnp.transpose(out_t, (0, 2, 1))


# speedup vs baseline: 1.1811x; 1.1811x over previous
import jax
import jax.numpy as jnp
from jax.experimental import pallas as pl
from jax.experimental.pallas import tpu as pltpu

TOK_BLK = 1024


def _router_kernel(x0_ref, x1_ref, w_ref, o_ref):
    w = w_ref[...]
    dims = (((1,), (1,)), ((), ()))
    o_ref[0, :, :TOK_BLK] = jax.lax.dot_general(
        w, x0_ref[0], dimension_numbers=dims,
        preferred_element_type=jnp.float32)
    o_ref[0, :, TOK_BLK:] = jax.lax.dot_general(
        w, x1_ref[0], dimension_numbers=dims,
        preferred_element_type=jnp.float32)


def kernel(hidden_states, W):
    B, S, H = hidden_states.shape
    E = W.shape[0]
    out_t = pl.pallas_call(
        _router_kernel,
        grid=(B, S // (2 * TOK_BLK)),
        in_specs=[
            pl.BlockSpec((1, TOK_BLK, H), lambda b, i: (b, 2 * i, 0)),
            pl.BlockSpec((1, TOK_BLK, H), lambda b, i: (b, 2 * i + 1, 0)),
            pl.BlockSpec((E, H), lambda b, i: (0, 0)),
        ],
        out_specs=pl.BlockSpec((1, E, 2 * TOK_BLK), lambda b, i: (b, 0, i)),
        out_shape=jax.ShapeDtypeStruct((B, E, S), jnp.float32),
        compiler_params=pltpu.CompilerParams(
            dimension_semantics=("arbitrary", "arbitrary"),
        ),
    )(hidden_states, hidden_states, W)
    return jnp.transpose(out_t, (0, 2, 1))


# speedup vs baseline: 1.1976x; 1.0140x over previous
import jax
import jax.numpy as jnp
from jax.experimental import pallas as pl
from jax.experimental.pallas import tpu as pltpu

TOK_BLK = 1024


def _router_kernel(x_ref, w_ref, o_ref):
    o_ref[0] = jax.lax.dot_general(
        w_ref[...], x_ref[0],
        dimension_numbers=(((1,), (1,)), ((), ())),
        preferred_element_type=jnp.float32)


def kernel(hidden_states, W):
    B, S, H = hidden_states.shape
    E = W.shape[0]
    out_t = pl.pallas_call(
        _router_kernel,
        grid=(B, S // TOK_BLK),
        in_specs=[
            pl.BlockSpec((1, TOK_BLK, H), lambda b, i: (b, i, 0)),
            pl.BlockSpec((E, H), lambda b, i: (0, 0)),
        ],
        out_specs=pl.BlockSpec((1, E, TOK_BLK), lambda b, i: (b, 0, i)),
        out_shape=jax.ShapeDtypeStruct((B, E, S), jnp.float32),
        compiler_params=pltpu.CompilerParams(
            dimension_semantics=("arbitrary", "arbitrary"),
        ),
    )(hidden_states, W)
    return jnp.transpose(out_t, (0, 2, 1))
